# baseline (speedup 1.0000x reference)
import functools

import jax
import jax.numpy as jnp
from jax import lax
from jax.experimental import pallas as pl
from jax.experimental.pallas import tpu as pltpu
from jax.experimental.pallas import tpu_sc as plsc

N = 2048
D_IN = 256
D = 256
D_HEAD = 64
H = 4
HALF = 64
KTOP = 32
TILE = 512
SLAB = 768
NT = N // TILE
SEG = 1024
INV_TEMP_SCALE = 0.25

_NW = 32
_ROWS_W = N // _NW
_CH = 8


def _body(x_ref, wq_ref, bq_ref, wk_ref, bk_ref, out_ref):
    t = pl.program_id(0)
    i0 = t * TILE
    seg_start = (t // (SEG // TILE)) * SEG
    s0 = pl.multiple_of(
        jnp.clip(TILE * t - 2 * HALF, seg_start, seg_start + SEG - SLAB), 128)

    dn = (((1,), (1,)), ((), ()))
    xq = x_ref[pl.ds(i0, TILE), :]
    q = lax.dot_general(xq, wq_ref[...], dn,
                        preferred_element_type=jnp.float32) + bq_ref[0, :][None, :]
    xk = x_ref[pl.ds(s0, SLAB), :]
    k = lax.dot_general(xk, wk_ref[...], dn,
                        preferred_element_type=jnp.float32) + bk_ref[0, :][None, :]

    cols = s0 + lax.broadcasted_iota(jnp.int32, (SLAB, TILE), 0)
    rows = i0 + lax.broadcasted_iota(jnp.int32, (SLAB, TILE), 1)
    valid = (cols >= rows - HALF) & (cols < rows + HALF)

    attn = jnp.zeros((SLAB, TILE), jnp.float32)
    for h in range(H):
        qh = q[:, h * D_HEAD:(h + 1) * D_HEAD]
        kh = k[:, h * D_HEAD:(h + 1) * D_HEAD]
        s = lax.dot_general(kh, qh, dn, preferred_element_type=jnp.float32,
                            precision=lax.Precision.HIGHEST)
        z = jnp.where(valid, s * INV_TEMP_SCALE, -1e30)
        m = jnp.max(z, axis=0, keepdims=True)
        p = jnp.exp(z - m)
        attn = attn + p / jnp.sum(p, axis=0, keepdims=True)
    attn = jnp.where(valid, attn * (1.0 / H), 0.0)

    u = lax.bitcast_convert_type(attn, jnp.int32)
    lo = jnp.zeros((1, TILE), jnp.int32)
    hi = jnp.full((1, TILE), 0x3D800000, jnp.int32)
    for _ in range(30):
        mid = lo + ((hi - lo + 1) >> 1)
        c = jnp.sum((u >= mid).astype(jnp.int32), axis=0, keepdims=True)
        ge = c >= KTOP
        lo = jnp.where(ge, mid, lo)
        hi = jnp.where(ge, hi, mid - 1)
    tau = lo
    gt = u > tau
    eq = u == tau
    cg = jnp.sum(gt.astype(jnp.float32), axis=0, keepdims=True)
    ii = lax.broadcasted_iota(jnp.int32, (SLAB, SLAB), 0)
    jj = lax.broadcasted_iota(jnp.int32, (SLAB, SLAB), 1)
    tri = (ii >= jj).astype(jnp.float32)
    rank_eq = lax.dot_general(tri, eq.astype(jnp.float32),
                              (((1,), (0,)), ((), ())),
                              preferred_element_type=jnp.float32)
    sel = gt | (eq & (rank_eq <= (KTOP - cg)))

    out_ref[...] = sel.astype(jnp.float32).T


def _tc_mask(x, wq, bq2, wk, bk2):
    return pl.pallas_call(
        _body,
        grid=(NT,),
        in_specs=[
            pl.BlockSpec((N, D_IN), lambda t: (0, 0)),
            pl.BlockSpec((D, D_IN), lambda t: (0, 0)),
            pl.BlockSpec((1, D), lambda t: (0, 0)),
            pl.BlockSpec((D, D_IN), lambda t: (0, 0)),
            pl.BlockSpec((1, D), lambda t: (0, 0)),
        ],
        out_specs=pl.BlockSpec((TILE, SLAB), lambda t: (t, 0)),
        out_shape=jax.ShapeDtypeStruct((N, SLAB), jnp.float32),
    )(x, wq, bq2, wk, bk2)


@functools.partial(
    pl.kernel,
    mesh=plsc.VectorSubcoreMesh(core_axis_name="c", subcore_axis_name="s"),
    out_type=jax.ShapeDtypeStruct((N, N), jnp.float32),
    scratch_types=[
        pltpu.VMEM((_CH, N), jnp.float32),
    ],
)
def _sc_assemble(zeros_hbm, mask_hbm, out_hbm, rowbuf):
    wid = lax.axis_index("s") * 2 + lax.axis_index("c")
    base = wid * _ROWS_W
    s0 = (wid // 16) * SEG + ((wid // 8) % 2) * 256
    pltpu.sync_copy(zeros_hbm, rowbuf)
    for cidx in range(_ROWS_W // _CH):
        r0 = base + cidx * _CH
        pltpu.sync_copy(mask_hbm.at[pl.ds(r0, _CH)],
                        rowbuf.at[:, pl.ds(s0, SLAB)])
        pltpu.sync_copy(rowbuf, out_hbm.at[pl.ds(r0, _CH)])


def kernel(node_features, Wq, bq, Wk, bk, Wv, bv, graph_num_nodes,
           num_relation):
    del Wv, bv, graph_num_nodes, num_relation
    bq2 = bq.reshape(1, D)
    bk2 = bk.reshape(1, D)
    mask = _tc_mask(node_features, Wq, bq2, Wk, bk2)
    zeros = jnp.zeros((_CH, N), jnp.float32)
    return _sc_assemble(zeros, mask)


# speedup vs baseline: 1.5154x; 1.5154x over previous
import jax
import jax.numpy as jnp
from jax import lax
from jax.experimental import pallas as pl

N = 2048
D_IN = 256
D = 256
D_HEAD = 64
H = 4
HALF = 64
KTOP = 32
TILE = 512
SLAB = 768
NT = N // TILE
SEG = 1024
INV_TEMP_SCALE = 0.25


def _body(x_ref, wq_ref, bq_ref, wk_ref, bk_ref, out_ref):
    t = pl.program_id(0)
    i0 = t * TILE
    seg_start = (t // (SEG // TILE)) * SEG
    s0 = pl.multiple_of(
        jnp.clip(TILE * t - 2 * HALF, seg_start, seg_start + SEG - SLAB), 128)

    dn = (((1,), (1,)), ((), ()))
    xq = x_ref[pl.ds(i0, TILE), :]
    q = lax.dot_general(xq, wq_ref[...], dn,
                        preferred_element_type=jnp.float32) + bq_ref[0, :][None, :]
    xk = x_ref[pl.ds(s0, SLAB), :]
    k = lax.dot_general(xk, wk_ref[...], dn,
                        preferred_element_type=jnp.float32) + bk_ref[0, :][None, :]

    cols = s0 + lax.broadcasted_iota(jnp.int32, (SLAB, TILE), 0)
    rows = i0 + lax.broadcasted_iota(jnp.int32, (SLAB, TILE), 1)
    valid = (cols >= rows - HALF) & (cols < rows + HALF)

    attn = jnp.zeros((SLAB, TILE), jnp.float32)
    for h in range(H):
        qh = q[:, h * D_HEAD:(h + 1) * D_HEAD]
        kh = k[:, h * D_HEAD:(h + 1) * D_HEAD]
        s = lax.dot_general(kh, qh, dn, preferred_element_type=jnp.float32,
                            precision=lax.Precision.HIGHEST)
        z = jnp.where(valid, s * INV_TEMP_SCALE, -1e30)
        m = jnp.max(z, axis=0, keepdims=True)
        p = jnp.exp(z - m)
        attn = attn + p / jnp.sum(p, axis=0, keepdims=True)
    attn = jnp.where(valid, attn * (1.0 / H), 0.0)

    u = lax.bitcast_convert_type(attn, jnp.int32)
    lo = jnp.zeros((1, TILE), jnp.int32)
    hi = jnp.full((1, TILE), 0x3D800000, jnp.int32)
    for _ in range(30):
        mid = lo + ((hi - lo + 1) >> 1)
        c = jnp.sum((u >= mid).astype(jnp.int32), axis=0, keepdims=True)
        ge = c >= KTOP
        lo = jnp.where(ge, mid, lo)
        hi = jnp.where(ge, hi, mid - 1)
    tau = lo
    gt = u > tau
    eq = u == tau
    cg = jnp.sum(gt.astype(jnp.float32), axis=0, keepdims=True)
    ii = lax.broadcasted_iota(jnp.int32, (SLAB, SLAB), 0)
    jj = lax.broadcasted_iota(jnp.int32, (SLAB, SLAB), 1)
    tri = (ii >= jj).astype(jnp.float32)
    rank_eq = lax.dot_general(tri, eq.astype(jnp.float32),
                              (((1,), (0,)), ((), ())),
                              preferred_element_type=jnp.float32)
    sel = gt | (eq & (rank_eq <= (KTOP - cg)))

    out_ref[...] = jnp.zeros((TILE, N), jnp.float32)
    out_ref[:, pl.ds(s0, SLAB)] = sel.astype(jnp.float32).T


def kernel(node_features, Wq, bq, Wk, bk, Wv, bv, graph_num_nodes,
           num_relation):
    del Wv, bv, graph_num_nodes, num_relation
    bq2 = bq.reshape(1, D)
    bk2 = bk.reshape(1, D)
    return pl.pallas_call(
        _body,
        grid=(NT,),
        in_specs=[
            pl.BlockSpec((N, D_IN), lambda t: (0, 0)),
            pl.BlockSpec((D, D_IN), lambda t: (0, 0)),
            pl.BlockSpec((1, D), lambda t: (0, 0)),
            pl.BlockSpec((D, D_IN), lambda t: (0, 0)),
            pl.BlockSpec((1, D), lambda t: (0, 0)),
        ],
        out_specs=pl.BlockSpec((TILE, N), lambda t: (t, 0)),
        out_shape=jax.ShapeDtypeStruct((N, N), jnp.float32),
    )(node_features, Wq, bq2, Wk, bk2)


# speedup vs baseline: 2.0489x; 1.3521x over previous
import jax
import jax.numpy as jnp
from jax import lax
from jax.experimental import pallas as pl

N = 2048
D_IN = 256
D = 256
D_HEAD = 64
H = 4
HALF = 64
KTOP = 32
TILE = 256
SLAB = 512
NT = N // TILE
SEG = 1024
INV_TEMP_SCALE = 0.25


def _body(x_ref, wq_ref, bq_ref, wk_ref, bk_ref, out_ref):
    t = pl.program_id(0)
    i0 = t * TILE
    seg_start = (t // (SEG // TILE)) * SEG
    s0 = pl.multiple_of(
        jnp.clip(TILE * t - 2 * HALF, seg_start, seg_start + SEG - SLAB), 128)

    dn = (((1,), (1,)), ((), ()))
    xq = x_ref[pl.ds(i0, TILE), :]
    q = lax.dot_general(xq, wq_ref[...], dn,
                        preferred_element_type=jnp.float32) + bq_ref[0, :][None, :]
    xk = x_ref[pl.ds(s0, SLAB), :]
    k = lax.dot_general(xk, wk_ref[...], dn,
                        preferred_element_type=jnp.float32) + bk_ref[0, :][None, :]

    cols = s0 + lax.broadcasted_iota(jnp.int32, (SLAB, TILE), 0)
    rows = i0 + lax.broadcasted_iota(jnp.int32, (SLAB, TILE), 1)
    valid = (cols >= rows - HALF) & (cols < rows + HALF)

    attn = jnp.zeros((SLAB, TILE), jnp.float32)
    for h in range(H):
        qh = q[:, h * D_HEAD:(h + 1) * D_HEAD]
        kh = k[:, h * D_HEAD:(h + 1) * D_HEAD]
        s = lax.dot_general(kh, qh, dn, preferred_element_type=jnp.float32,
                            precision=lax.Precision.HIGHEST)
        z = jnp.where(valid, s * INV_TEMP_SCALE, -1e30)
        m = jnp.max(z, axis=0, keepdims=True)
        p = jnp.exp(z - m)
        attn = attn + p / jnp.sum(p, axis=0, keepdims=True)
    attn = jnp.where(valid, attn * (1.0 / H), 0.0)

    u = lax.bitcast_convert_type(attn, jnp.int32)
    lo = jnp.zeros((1, TILE), jnp.int32)
    hi = jnp.full((1, TILE), 0x3D800000, jnp.int32)
    for _ in range(30):
        mid = lo + ((hi - lo + 1) >> 1)
        c = jnp.sum((u >= mid).astype(jnp.int32), axis=0, keepdims=True)
        ge = c >= KTOP
        lo = jnp.where(ge, mid, lo)
        hi = jnp.where(ge, hi, mid - 1)
    tau = lo
    gt = u > tau
    eq = u == tau
    cg = jnp.sum(gt.astype(jnp.float32), axis=0, keepdims=True)
    ii = lax.broadcasted_iota(jnp.int32, (SLAB, SLAB), 0)
    jj = lax.broadcasted_iota(jnp.int32, (SLAB, SLAB), 1)
    tri = (ii >= jj).astype(jnp.float32)
    rank_eq = lax.dot_general(tri, eq.astype(jnp.float32),
                              (((1,), (0,)), ((), ())),
                              preferred_element_type=jnp.float32)
    sel = gt | (eq & (rank_eq <= (KTOP - cg)))

    out_ref[...] = jnp.zeros((TILE, N), jnp.float32)
    out_ref[:, pl.ds(s0, SLAB)] = sel.astype(jnp.float32).T


def kernel(node_features, Wq, bq, Wk, bk, Wv, bv, graph_num_nodes,
           num_relation):
    del Wv, bv, graph_num_nodes, num_relation
    bq2 = bq.reshape(1, D)
    bk2 = bk.reshape(1, D)
    return pl.pallas_call(
        _body,
        grid=(NT,),
        in_specs=[
            pl.BlockSpec((N, D_IN), lambda t: (0, 0)),
            pl.BlockSpec((D, D_IN), lambda t: (0, 0)),
            pl.BlockSpec((1, D), lambda t: (0, 0)),
            pl.BlockSpec((D, D_IN), lambda t: (0, 0)),
            pl.BlockSpec((1, D), lambda t: (0, 0)),
        ],
        out_specs=pl.BlockSpec((TILE, N), lambda t: (t, 0)),
        out_shape=jax.ShapeDtypeStruct((N, N), jnp.float32),
    )(node_features, Wq, bq2, Wk, bk2)
